# baseline (speedup 1.0000x reference)
import jax
import jax.numpy as jnp
from jax import lax
from jax.experimental import pallas as pl
from jax.experimental.pallas import tpu as pltpu
from jax.experimental.pallas import tpu_sc as plsc

N = 10000
E = 320000
H = 128
L = 16
NC = 2
NS = 16
NW = NC * NS
EPW = E // NW
C = 80
NCHUNK = EPW // C
NBUF = 4


def _pre_body(zs_ref, zd_ref, w1_ref, b1_ref, w2_ref, b2_ref,
              a_ref, b_ref, w2o_ref, b2o_ref):
    dn = (((1,), (1,)), ((), ()))
    w1 = w1_ref[...]
    a_ref[...] = lax.dot_general(zs_ref[...], w1[:, :H], dn,
                                 preferred_element_type=jnp.float32)
    b_ref[...] = lax.dot_general(zd_ref[...], w1[:, H:], dn,
                                 preferred_element_type=jnp.float32) + b1_ref[...]
    w2o_ref[...] = w2_ref[...]
    b2o_ref[...] = jnp.broadcast_to(b2_ref[...], (1, L))


def _lane_shuffle(v, perm):
    dnums = lax.GatherDimensionNumbers(
        offset_dims=(), collapsed_slice_dims=(0,), start_index_map=(0,))
    return lax.gather(v, perm[:, None], dnums, (1,),
                      mode=lax.GatherScatterMode.PROMISE_IN_BOUNDS)


def _edge_body(a_hbm, b_hbm, eli_hbm, w2_hbm, b2_hbm, out_hbm,
               idx_r, idx_c, srows, outv, w2v, b2v, semA, semB):
    wid = lax.axis_index("s") * NC + lax.axis_index("c")

    pltpu.sync_copy(eli_hbm.at[0, wid], idx_r)
    pltpu.sync_copy(eli_hbm.at[1, wid], idx_c)
    pltpu.sync_copy(w2_hbm.at[0], w2v)
    pltpu.sync_copy(b2_hbm.at[0], b2v)

    w2regs = [w2v[pl.ds(L * k, L)] for k in range(H // L)]
    b2reg = b2v[...]
    iota16 = lax.iota(jnp.int32, L)
    perms = {s: iota16 ^ s for s in (1, 2, 4, 8)}
    masks = {s: (iota16 & s) == 0 for s in (1, 2, 4, 8)}
    zeros16 = jnp.zeros((L,), jnp.float32)

    def fire(i, buf):
        pltpu.async_copy(a_hbm.at[idx_r.at[i]], srows.at[buf], semA, add=True)
        pltpu.async_copy(b_hbm.at[idx_c.at[i]], srows.at[buf], semB, add=True)

    def wait(i, buf):
        pltpu.make_async_copy(a_hbm.at[idx_r.at[i]], srows.at[buf], semA).wait()
        pltpu.make_async_copy(b_hbm.at[idx_c.at[i]], srows.at[buf], semB).wait()

    def zero_body(j, carry):
        for b in range(NBUF):
            for k in range(H // L):
                srows[b, j, pl.ds(k * L, L)] = zeros16
        return carry

    lax.fori_loop(0, C, zero_body, 0)
    for b in range(NBUF):
        fire(b, b)

    def chunk_body(i, carry):
        buf = lax.rem(i, NBUF)
        wait(i, buf)

        @plsc.parallel_loop(0, C // L, unroll=1)
        def group_body(g):
            vs = []
            for e16 in range(L):
                e = g * L + e16
                acc = None
                for k in range(H // L):
                    s = srows[buf, e, pl.ds(k * L, L)]
                    t = jnp.maximum(s, 0.0) * w2regs[k]
                    acc = t if acc is None else acc + t
                    srows[buf, e, pl.ds(k * L, L)] = zeros16
                vs.append(acc)
            for s in (1, 2, 4, 8):
                m, p = masks[s], perms[s]
                vs = [jnp.where(m, a, _lane_shuffle(b, p))
                      + jnp.where(m, _lane_shuffle(a, p), b)
                      for a, b in zip(vs[0::2], vs[1::2])]
            zv = vs[0] + b2reg
            sig = 1.0 / (1.0 + jnp.exp(-zv))
            outv[pl.ds(i * C + g * L, L)] = sig

        @pl.when(i + NBUF < NCHUNK)
        def _():
            fire(i + NBUF, buf)

        return carry

    lax.fori_loop(0, NCHUNK, chunk_body, 0)
    pltpu.sync_copy(outv, out_hbm.at[pl.ds(wid * EPW, EPW)])


@jax.jit
def kernel(z_src, z_dst, edge_label_index, W1, b1, W2, b2):
    eli4 = edge_label_index.reshape(2, NW, NCHUNK, C)
    b1r = b1.reshape(1, H)
    b2r = b2.reshape(1, 1)

    a_nodes, b_nodes, w2o, b2o = pl.pallas_call(
        _pre_body,
        out_shape=[
            jax.ShapeDtypeStruct((N, H), jnp.float32),
            jax.ShapeDtypeStruct((N, H), jnp.float32),
            jax.ShapeDtypeStruct((1, H), jnp.float32),
            jax.ShapeDtypeStruct((1, L), jnp.float32),
        ],
    )(z_src, z_dst, W1, b1r, W2, b2r)

    edge_kernel = pl.kernel(
        _edge_body,
        out_type=jax.ShapeDtypeStruct((E,), jnp.float32),
        mesh=plsc.VectorSubcoreMesh(
            core_axis_name="c", subcore_axis_name="s",
            num_cores=NC, num_subcores=NS),
        scratch_types=[
            pltpu.VMEM((NCHUNK, C), jnp.int32),
            pltpu.VMEM((NCHUNK, C), jnp.int32),
            pltpu.VMEM((NBUF, C, H), jnp.float32),
            pltpu.VMEM((EPW,), jnp.float32),
            pltpu.VMEM((H,), jnp.float32),
            pltpu.VMEM((L,), jnp.float32),
            pltpu.SemaphoreType.DMA,
            pltpu.SemaphoreType.DMA,
        ],
    )
    return edge_kernel(a_nodes, b_nodes, eli4, w2o, b2o)


# speedup vs baseline: 1.0127x; 1.0127x over previous
import jax
import jax.numpy as jnp
from jax import lax
from jax.experimental import pallas as pl
from jax.experimental.pallas import tpu as pltpu
from jax.experimental.pallas import tpu_sc as plsc

N = 10000
E = 320000
H = 128
L = 16
NC = 2
NS = 16
NW = NC * NS
EPW = E // NW
C = 80
NCHUNK = EPW // C
NBUF = 4


def _pre_body(zs_ref, zd_ref, w1_ref, b1_ref, w2_ref, b2_ref,
              a_ref, b_ref, w2o_ref, b2o_ref):
    dn = (((1,), (1,)), ((), ()))
    w1 = w1_ref[...]
    a_ref[...] = lax.dot_general(zs_ref[...], w1[:, :H], dn,
                                 preferred_element_type=jnp.float32)
    b_ref[...] = lax.dot_general(zd_ref[...], w1[:, H:], dn,
                                 preferred_element_type=jnp.float32) + b1_ref[...]
    w2o_ref[...] = w2_ref[...]
    b2o_ref[...] = jnp.broadcast_to(b2_ref[...], (1, L))


def _lane_shuffle(v, perm):
    dnums = lax.GatherDimensionNumbers(
        offset_dims=(), collapsed_slice_dims=(0,), start_index_map=(0,))
    return lax.gather(v, perm[:, None], dnums, (1,),
                      mode=lax.GatherScatterMode.PROMISE_IN_BOUNDS)


def _edge_body(a_hbm, b_hbm, eli_hbm, w2_hbm, b2_hbm, out_hbm,
               idx_r, idx_c, srows, outv, w2v, b2v, semA, semB):
    wid = lax.axis_index("s") * NC + lax.axis_index("c")

    pltpu.sync_copy(eli_hbm.at[0, wid], idx_r)
    pltpu.sync_copy(eli_hbm.at[1, wid], idx_c)
    pltpu.sync_copy(w2_hbm.at[0], w2v)
    pltpu.sync_copy(b2_hbm.at[0], b2v)

    w2regs = [w2v[pl.ds(L * k, L)] for k in range(H // L)]
    b2reg = b2v[...]
    iota16 = lax.iota(jnp.int32, L)
    perms = {s: iota16 ^ s for s in (1, 2, 4, 8)}
    masks = {s: (iota16 & s) == 0 for s in (1, 2, 4, 8)}
    zeros16 = jnp.zeros((L,), jnp.float32)

    def fire(i, buf):
        pltpu.async_copy(a_hbm.at[idx_r.at[i]], srows.at[buf], semA, add=True)
        pltpu.async_copy(b_hbm.at[idx_c.at[i]], srows.at[buf], semB, add=True)

    def wait(i, buf):
        pltpu.make_async_copy(a_hbm.at[idx_r.at[i]], srows.at[buf], semA).wait()
        pltpu.make_async_copy(b_hbm.at[idx_c.at[i]], srows.at[buf], semB).wait()

    def zero_body(j, carry):
        for b in range(NBUF):
            for k in range(H // L):
                srows[b, j, pl.ds(k * L, L)] = zeros16
        return carry

    lax.fori_loop(0, C, zero_body, 0)
    for b in range(NBUF):
        fire(b, b)

    def chunk_body(i, carry):
        buf = lax.rem(i, NBUF)
        wait(i, buf)

        def group_body(g, gcarry):
            vs = []
            for e16 in range(L):
                e = g * L + e16
                acc = None
                for k in range(H // L):
                    s = srows[buf, e, pl.ds(k * L, L)]
                    t = jnp.maximum(s, 0.0) * w2regs[k]
                    acc = t if acc is None else acc + t
                    srows[buf, e, pl.ds(k * L, L)] = zeros16
                vs.append(acc)
            for s in (1, 2, 4, 8):
                m, p = masks[s], perms[s]
                vs = [jnp.where(m, a, _lane_shuffle(b, p))
                      + jnp.where(m, _lane_shuffle(a, p), b)
                      for a, b in zip(vs[0::2], vs[1::2])]
            zv = vs[0] + b2reg
            sig = 1.0 / (1.0 + jnp.exp(-zv))
            outv[pl.ds(i * C + g * L, L)] = sig
            return gcarry

        lax.fori_loop(0, C // L, group_body, 0)

        @pl.when(i + NBUF < NCHUNK)
        def _():
            fire(i + NBUF, buf)

        return carry

    lax.fori_loop(0, NCHUNK, chunk_body, 0)
    pltpu.sync_copy(outv, out_hbm.at[pl.ds(wid * EPW, EPW)])


@jax.jit
def kernel(z_src, z_dst, edge_label_index, W1, b1, W2, b2):
    eli4 = edge_label_index.reshape(2, NW, NCHUNK, C)
    b1r = b1.reshape(1, H)
    b2r = b2.reshape(1, 1)

    a_nodes, b_nodes, w2o, b2o = pl.pallas_call(
        _pre_body,
        out_shape=[
            jax.ShapeDtypeStruct((N, H), jnp.float32),
            jax.ShapeDtypeStruct((N, H), jnp.float32),
            jax.ShapeDtypeStruct((1, H), jnp.float32),
            jax.ShapeDtypeStruct((1, L), jnp.float32),
        ],
    )(z_src, z_dst, W1, b1r, W2, b2r)

    edge_kernel = pl.kernel(
        _edge_body,
        out_type=jax.ShapeDtypeStruct((E,), jnp.float32),
        mesh=plsc.VectorSubcoreMesh(
            core_axis_name="c", subcore_axis_name="s",
            num_cores=NC, num_subcores=NS),
        scratch_types=[
            pltpu.VMEM((NCHUNK, C), jnp.int32),
            pltpu.VMEM((NCHUNK, C), jnp.int32),
            pltpu.VMEM((NBUF, C, H), jnp.float32),
            pltpu.VMEM((EPW,), jnp.float32),
            pltpu.VMEM((H,), jnp.float32),
            pltpu.VMEM((L,), jnp.float32),
            pltpu.SemaphoreType.DMA,
            pltpu.SemaphoreType.DMA,
        ],
    )
    return edge_kernel(a_nodes, b_nodes, eli4, w2o, b2o)
